# baseline (speedup 1.0000x reference)
import functools

import jax
import jax.numpy as jnp
from jax import lax
from jax.experimental import pallas as pl
from jax.experimental.pallas import tpu as pltpu
from jax.experimental.pallas import tpu_sc as plsc

N = 100000
M = 800000
D = 64
M2 = M // 2

NW = 32
PER_W = M // NW
CHUNK = 200
NB = 5
NCH = PER_W // CHUNK
NG = NCH // NB

BH = 3200
NTB = M2 // BH

_mesh = plsc.VectorSubcoreMesh(core_axis_name="c", subcore_axis_name="s")


@functools.partial(
    pl.kernel,
    mesh=_mesh,
    out_type=jax.ShapeDtypeStruct((M, D), jnp.float32),
    scratch_types=[
        pltpu.VMEM((PER_W,), jnp.int32),
        *[pltpu.VMEM((CHUNK, D), jnp.float32) for _ in range(NB)],
        *[pltpu.SemaphoreType.DMA for _ in range(2 * NB)],
    ],
    compiler_params=pltpu.CompilerParams(use_tc_tiling_on_sc=False),
)
def _sc_gather(feats_hbm, idx_hbm, out_hbm, idx_v, *bufs):
    rows = bufs[:NB]
    gsem = bufs[NB : 2 * NB]
    ssem = bufs[2 * NB :]

    c = lax.axis_index("c")
    s = lax.axis_index("s")
    wid = s * 2 + c
    base = pl.multiple_of(wid * PER_W, 8)
    pltpu.sync_copy(idx_hbm.at[pl.ds(base, PER_W)], idx_v)

    def group(g, _):
        gh = []
        for b in range(NB):
            off = pl.multiple_of((g * NB + b) * CHUNK, 8)
            gh.append(
                pltpu.async_copy(
                    feats_hbm.at[idx_v.at[pl.ds(off, CHUNK)]], rows[b], gsem[b]
                )
            )
        sh = []
        for b in range(NB):
            off = pl.multiple_of(base + (g * NB + b) * CHUNK, 8)
            gh[b].wait()
            sh.append(
                pltpu.async_copy(rows[b], out_hbm.at[pl.ds(off, CHUNK)], ssem[b])
            )
        for h in sh:
            h.wait()
        return 0

    lax.fori_loop(0, NG, group, 0)


def _tr_body(x_ref, o_ref):
    a = x_ref[...]
    o_ref[:, 0, :] = a[:, :D].T
    o_ref[:, 1, :] = a[:, D:].T


_tc_transpose = pl.pallas_call(
    _tr_body,
    grid=(NTB,),
    in_specs=[pl.BlockSpec((BH, 2 * D), lambda i: (i, 0))],
    out_specs=pl.BlockSpec((D, 2, BH), lambda i: (0, 0, i)),
    out_shape=jax.ShapeDtypeStruct((D, 2, M2), jnp.float32),
)


def kernel(feats, idx):
    idx32 = idx.astype(jnp.int32)
    idx_i = jnp.stack([idx32[:M2], idx32[M2:]], axis=1).reshape(-1)
    out_lin = _sc_gather(feats, idx_i)
    x128 = out_lin.reshape(M2, 2 * D)
    out_t3 = _tc_transpose(x128)
    return out_t3.reshape(D, M).T
